# Initial kernel scaffold; baseline (speedup 1.0000x reference)
import jax
import jax.numpy as jnp
from jax.experimental import pallas as pl


def kernel(x, edge_attr, emb_table, params, edge_index, fidelity_indicator, batch_mapping):
    raise NotImplementedError("write your pallas kernel here")


# speedup vs baseline: 1.1943x; 1.1943x over previous
import functools

import jax
import jax.numpy as jnp
from jax import lax
from jax.experimental import pallas as pl
from jax.experimental.pallas import tpu as pltpu
from jax.experimental.pallas import tpu_sc as plsc

_info = plsc.get_sparse_core_info()
_NC = _info.num_cores
_NS = _info.num_subcores
_NW = _NC * _NS

_C = 80


def _make_sc_gather(R, M, D):
  nch = M // _C
  ceil = (nch + _NW - 1) // _NW
  mesh = plsc.VectorSubcoreMesh(core_axis_name="c", subcore_axis_name="s")

  @functools.partial(
      pl.kernel, mesh=mesh,
      out_type=jax.ShapeDtypeStruct((M, D), jnp.float32),
      scratch_types=[
          pltpu.VMEM((_C,), jnp.int32),
          pltpu.VMEM((_C, D), jnp.float32),
          pltpu.SemaphoreType.DMA,
      ])
  def k(table, idx, out, idx_v, rows_v, sem):
    wid = lax.axis_index("s") * _NC + lax.axis_index("c")

    def body(i, carry):
      ch = i * _NW + wid

      @pl.when(ch < nch)
      def _():
        base = pl.multiple_of(ch * _C, 8)
        pltpu.sync_copy(idx.at[pl.ds(base, _C)], idx_v)
        pltpu.async_copy(table.at[idx_v], rows_v, sem).wait()
        pltpu.sync_copy(rows_v, out.at[pl.ds(base, _C)])

      return carry

    lax.fori_loop(0, ceil, body, 0)

  return k


def _make_sc_scatter_add(M, S, D):
  nch = M // _C
  ceil = (nch + _NW - 1) // _NW
  s_nch = S // _C
  s_ceil = (s_nch + _NS - 1) // _NS
  mesh = plsc.VectorSubcoreMesh(core_axis_name="c", subcore_axis_name="s")

  @functools.partial(
      pl.kernel, mesh=mesh,
      out_type=jax.ShapeDtypeStruct((_NC, S, D), jnp.float32),
      scratch_types=[
          pltpu.VMEM((_C,), jnp.int32),
          pltpu.VMEM((_C, D), jnp.float32),
          pltpu.VMEM_SHARED((S, D), jnp.float32),
      ])
  def k(vals, idx, zeros, out, idx_v, vals_v, shared):
    cid = lax.axis_index("c")
    sid = lax.axis_index("s")
    wid = sid * _NC + cid

    def zbody(i, carry):
      ch = i * _NS + sid

      @pl.when(ch < s_nch)
      def _():
        base = pl.multiple_of(ch * _C, 8)
        pltpu.sync_copy(zeros, shared.at[pl.ds(base, _C)])

      return carry

    lax.fori_loop(0, s_ceil, zbody, 0)
    plsc.subcore_barrier()

    def body(i, carry):
      ch = i * _NW + wid

      @pl.when(ch < nch)
      def _():
        base = pl.multiple_of(ch * _C, 8)
        pltpu.sync_copy(idx.at[pl.ds(base, _C)], idx_v)
        pltpu.sync_copy(vals.at[pl.ds(base, _C)], vals_v)
        pltpu.sync_copy(vals_v, shared.at[idx_v], add=True)

      return carry

    lax.fori_loop(0, ceil, body, 0)
    plsc.subcore_barrier()

    def obody(i, carry):
      ch = i * _NS + sid

      @pl.when(ch < s_nch)
      def _():
        base = pl.multiple_of(ch * _C, 8)
        pltpu.sync_copy(shared.at[pl.ds(base, _C)], out.at[cid, pl.ds(base, _C)])

      return carry

    lax.fori_loop(0, s_ceil, obody, 0)

  return k


_DOT = dict(precision=lax.Precision.HIGHEST, preferred_element_type=jnp.float32)


def _onehot(ids2d, B):
  return (ids2d == lax.broadcasted_iota(jnp.int32, (1, B), 1)).astype(jnp.float32)


def _edge_mlp_body(relu_ea, B,
                   gxs, gxd, ea, ids, fid, w1a, w1b, w1c, w1d, b1, w2, b2,
                   ue, esum, ecnt):
  pid = pl.program_id(0)

  @pl.when(pid == 0)
  def _():
    esum[...] = jnp.zeros_like(esum)
    ecnt[...] = jnp.zeros_like(ecnt)

  ea_t = ea[...]
  if relu_ea:
    ea_t = jnp.maximum(ea_t, 0.0)
  oh = _onehot(ids[...], B)
  t = jnp.dot(fid[...], w1d[...], **_DOT)
  h = (jnp.dot(gxs[...], w1a[...], **_DOT)
       + jnp.dot(gxd[...], w1b[...], **_DOT)
       + jnp.dot(ea_t, w1c[...], **_DOT)
       + jnp.dot(oh, t, **_DOT) + b1[...])
  h = jnp.maximum(h, 0.0)
  u = jnp.dot(h, w2[...], **_DOT) + b2[...]
  ue[...] = u
  cdims = (((0,), (0,)), ((), ()))
  esum[...] += lax.dot_general(oh, u, cdims, **_DOT)
  ecnt[...] += lax.dot_general(oh, jnp.ones_like(ids[...], jnp.float32), cdims,
                               **_DOT)


def _edge_mlp(gxs, gxd, ea, ids, fid, w1a, w1b, w1c, w1d, b1, w2, b2,
              relu_ea, tile):
  E = gxs.shape[0]
  Dea = ea.shape[1]
  B = fid.shape[0]
  grid = E // tile
  full = lambda a: pl.BlockSpec(a.shape, lambda i: (0,) * a.ndim)
  return pl.pallas_call(
      functools.partial(_edge_mlp_body, relu_ea, B),
      grid=(grid,),
      in_specs=[
          pl.BlockSpec((tile, 128), lambda i: (i, 0)),
          pl.BlockSpec((tile, 128), lambda i: (i, 0)),
          pl.BlockSpec((tile, Dea), lambda i: (i, 0)),
          pl.BlockSpec((tile, 1), lambda i: (i, 0)),
          full(fid), full(w1a), full(w1b), full(w1c), full(w1d), full(b1),
          full(w2), full(b2),
      ],
      out_specs=[
          pl.BlockSpec((tile, 128), lambda i: (i, 0)),
          pl.BlockSpec((B, 128), lambda i: (0, 0)),
          pl.BlockSpec((B, 1), lambda i: (0, 0)),
      ],
      out_shape=[
          jax.ShapeDtypeStruct((E, 128), jnp.float32),
          jax.ShapeDtypeStruct((B, 128), jnp.float32),
          jax.ShapeDtypeStruct((B, 1), jnp.float32),
      ],
  )(gxs, gxd, ea, ids, fid, w1a, w1b, w1c, w1d, b1, w2, b2)


def _node_mlp_body(B, sums, cnts, xn, ids, fid, w1a, w1b, w1c, b1, w2, b2,
                   xnext, nsum, ncnt):
  pid = pl.program_id(0)

  @pl.when(pid == 0)
  def _():
    nsum[...] = jnp.zeros_like(nsum)
    ncnt[...] = jnp.zeros_like(ncnt)

  avg = sums[...] / jnp.maximum(cnts[...], 1.0)
  oh = _onehot(ids[...], B)
  t = jnp.dot(fid[...], w1c[...], **_DOT)
  h = (jnp.dot(avg, w1a[...], **_DOT)
       + jnp.dot(xn[...], w1b[...], **_DOT)
       + jnp.dot(oh, t, **_DOT) + b1[...])
  h = jnp.maximum(h, 0.0)
  u = jnp.dot(h, w2[...], **_DOT) + b2[...]
  xnext[...] = jnp.maximum(u, 0.0)
  cdims = (((0,), (0,)), ((), ()))
  nsum[...] += lax.dot_general(oh, u, cdims, **_DOT)
  ncnt[...] += lax.dot_general(oh, jnp.ones_like(ids[...], jnp.float32), cdims,
                               **_DOT)


def _node_mlp(sums, cnts, xn, ids, fid, w1a, w1b, w1c, b1, w2, b2, tile):
  N = xn.shape[0]
  B = fid.shape[0]
  grid = N // tile
  full = lambda a: pl.BlockSpec(a.shape, lambda i: (0,) * a.ndim)
  return pl.pallas_call(
      functools.partial(_node_mlp_body, B),
      grid=(grid,),
      in_specs=[
          pl.BlockSpec((tile, 128), lambda i: (i, 0)),
          pl.BlockSpec((tile, 128), lambda i: (i, 0)),
          pl.BlockSpec((tile, 128), lambda i: (i, 0)),
          pl.BlockSpec((tile, 1), lambda i: (i, 0)),
          full(fid), full(w1a), full(w1b), full(w1c), full(b1), full(w2),
          full(b2),
      ],
      out_specs=[
          pl.BlockSpec((tile, 128), lambda i: (i, 0)),
          pl.BlockSpec((B, 128), lambda i: (0, 0)),
          pl.BlockSpec((B, 1), lambda i: (0, 0)),
      ],
      out_shape=[
          jax.ShapeDtypeStruct((N, 128), jnp.float32),
          jax.ShapeDtypeStruct((B, 128), jnp.float32),
          jax.ShapeDtypeStruct((B, 1), jnp.float32),
      ],
  )(sums, cnts, xn, ids, fid, w1a, w1b, w1c, b1, w2, b2)


def _fid_mlp_body(nsum, ncnt, esum, ecnt, fid, w1a, w1b, w1c, b1, w2, b2, out):
  avg_n = nsum[...] / jnp.maximum(ncnt[...], 1.0)
  avg_e = esum[...] / jnp.maximum(ecnt[...], 1.0)
  h = (jnp.dot(avg_n, w1a[...], **_DOT)
       + jnp.dot(avg_e, w1b[...], **_DOT)
       + jnp.dot(fid[...], w1c[...], **_DOT) + b1[...])
  h = jnp.maximum(h, 0.0)
  out[...] = jnp.dot(h, w2[...], **_DOT) + b2[...]


def _fid_mlp(nsum, ncnt, esum, ecnt, fid, w1a, w1b, w1c, b1, w2, b2):
  B = fid.shape[0]
  Do = w2.shape[1]
  return pl.pallas_call(
      _fid_mlp_body,
      out_shape=jax.ShapeDtypeStruct((B, Do), jnp.float32),
  )(nsum, ncnt, esum, ecnt, fid, w1a, w1b, w1c, b1, w2, b2)


def _pool_body(relu_in, B, vals, ids, psum, pcnt, pmax):
  pid = pl.program_id(0)

  @pl.when(pid == 0)
  def _():
    psum[...] = jnp.zeros_like(psum)
    pcnt[...] = jnp.zeros_like(pcnt)
    pmax[...] = jnp.full_like(pmax, -jnp.inf)

  v = vals[...]
  if relu_in:
    v = jnp.maximum(v, 0.0)
  ids_t = ids[...]
  oh = _onehot(ids_t, B)
  cdims = (((0,), (0,)), ((), ()))
  psum[...] += lax.dot_general(oh, v, cdims, **_DOT)
  pcnt[...] += lax.dot_general(oh, jnp.ones_like(ids_t, jnp.float32), cdims,
                               **_DOT)
  rows = []
  for b in range(B):
    mask = ids_t == b
    rows.append(jnp.max(jnp.where(mask, v, -jnp.inf), axis=0, keepdims=True))
  pmax[...] = jnp.maximum(pmax[...], jnp.concatenate(rows, axis=0))


def _pool(vals, ids, relu_in, B, tile):
  M = vals.shape[0]
  grid = M // tile
  return pl.pallas_call(
      functools.partial(_pool_body, relu_in, B),
      grid=(grid,),
      in_specs=[
          pl.BlockSpec((tile, 128), lambda i: (i, 0)),
          pl.BlockSpec((tile, 1), lambda i: (i, 0)),
      ],
      out_specs=[
          pl.BlockSpec((B, 128), lambda i: (0, 0)),
          pl.BlockSpec((B, 1), lambda i: (0, 0)),
          pl.BlockSpec((B, 128), lambda i: (0, 0)),
      ],
      out_shape=[
          jax.ShapeDtypeStruct((B, 128), jnp.float32),
          jax.ShapeDtypeStruct((B, 1), jnp.float32),
          jax.ShapeDtypeStruct((B, 128), jnp.float32),
      ],
  )(vals, ids)


def kernel(x, edge_attr, emb_table, params, edge_index, fidelity_indicator,
           batch_mapping):
  N = x.shape[0]
  E = edge_attr.shape[0]
  B = fidelity_indicator.shape[0]
  src = edge_index[0].astype(jnp.int32)
  dst = edge_index[1].astype(jnp.int32)
  ebatch = batch_mapping[src].astype(jnp.int32)
  batch2 = batch_mapping.astype(jnp.int32).reshape(N, 1)
  ebatch2 = ebatch.reshape(E, 1)
  fid = emb_table[fidelity_indicator]

  gather_n = _make_sc_gather(N, E, 128)
  scatter_n = _make_sc_scatter_add(E, N, 128)
  zeros_c = jnp.zeros((_C, 128), jnp.float32)

  cnt2 = scatter_n(jnp.ones((E, 128), jnp.float32), dst, zeros_c)
  cnts = cnt2[0] + cnt2[1]

  def split(pre, sizes):
    w1 = params[pre + "_W1"]
    parts = []
    o = 0
    for s in sizes:
      parts.append(w1[o:o + s])
      o += s
    return parts, params[pre + "_b1"].reshape(1, -1), params[pre + "_W2"], \
        params[pre + "_b2"].reshape(1, -1)

  xl = x
  el = edge_attr
  fl = fid
  f_raw = None
  ue = None
  for li, pre in enumerate(["m0", "m1", "m2"]):
    dea = el.shape[1]
    (w1a, w1b, w1c, w1d), b1e, w2e, b2e = split(pre + "_e",
                                                [128, 128, dea, 32])
    gxs = gather_n(xl, src)
    gxd = gather_n(xl, dst)
    ue, esum, ecnt = _edge_mlp(gxs, gxd, el, ebatch2, fl,
                               w1a, w1b, w1c, w1d, b1e, w2e, b2e,
                               relu_ea=(li > 0), tile=640)
    sum2 = scatter_n(ue, dst, zeros_c)
    sums = sum2[0] + sum2[1]
    (v1a, v1b, v1c), b1v, w2v, b2v = split(pre + "_v", [128, 128, 32])
    xl, nsum, ncnt = _node_mlp(sums, cnts, xl, batch2, fl,
                               v1a, v1b, v1c, b1v, w2v, b2v, tile=400)
    (u1a, u1b, u1c), b1u, w2u, b2u = split(pre + "_u", [128, 128, 32])
    f_raw = _fid_mlp(nsum, ncnt, esum, ecnt, fl,
                     u1a, u1b, u1c, b1u, w2u, b2u)
    fl = jnp.maximum(f_raw, 0.0)
    el = ue

  xs, xc, xm = _pool(xl, batch2, False, B, tile=400)
  es, ec, em = _pool(ue, ebatch2, True, B, tile=640)
  xmean = xs / jnp.maximum(xc, 1.0)
  emean = es / jnp.maximum(ec, 1.0)
  xm = jnp.where(xc > 0, xm, 0.0)
  em = jnp.where(ec > 0, em, 0.0)
  x_batched = jnp.concatenate([xs, xmean, xm], axis=1)
  e_batched = jnp.concatenate([es, emean, em], axis=1)
  return (x_batched, e_batched, fl)


# speedup vs baseline: 1.6207x; 1.3570x over previous
import functools

import jax
import jax.numpy as jnp
from jax import lax
from jax.experimental import pallas as pl
from jax.experimental.pallas import tpu as pltpu
from jax.experimental.pallas import tpu_sc as plsc

_info = plsc.get_sparse_core_info()
_NC = _info.num_cores
_NS = _info.num_subcores
_NW = _NC * _NS

_C = 80


def _make_sc_gather(R, M, D):
  nch = M // _C
  ceil = (nch + _NW - 1) // _NW
  mesh = plsc.VectorSubcoreMesh(core_axis_name="c", subcore_axis_name="s")

  @functools.partial(
      pl.kernel, mesh=mesh,
      out_type=jax.ShapeDtypeStruct((M, D), jnp.float32),
      scratch_types=[
          pltpu.VMEM((_C,), jnp.int32),
          pltpu.VMEM((_C, D), jnp.float32),
          pltpu.SemaphoreType.DMA,
      ])
  def k(table, idx, out, idx_v, rows_v, sem):
    wid = lax.axis_index("s") * _NC + lax.axis_index("c")

    def body(i, carry):
      ch = i * _NW + wid

      @pl.when(ch < nch)
      def _():
        base = pl.multiple_of(ch * _C, 8)
        pltpu.sync_copy(idx.at[pl.ds(base, _C)], idx_v)
        pltpu.async_copy(table.at[idx_v], rows_v, sem).wait()
        pltpu.sync_copy(rows_v, out.at[pl.ds(base, _C)])

      return carry

    lax.fori_loop(0, ceil, body, 0)

  return k


def _make_sc_scatter_add(M, S, D, ones_mode=False):
  nch = M // _C
  ceil = (nch + _NW - 1) // _NW
  s_nch = S // _C
  s_ceil = (s_nch + _NS - 1) // _NS
  mesh = plsc.VectorSubcoreMesh(core_axis_name="c", subcore_axis_name="s")

  @functools.partial(
      pl.kernel, mesh=mesh,
      out_type=jax.ShapeDtypeStruct((_NC, S, D), jnp.float32),
      scratch_types=[
          pltpu.VMEM((_C,), jnp.int32),
          pltpu.VMEM((_C, D), jnp.float32),
          pltpu.VMEM_SHARED((S, D), jnp.float32),
      ])
  def k(vals, idx, zeros, out, idx_v, vals_v, shared):
    cid = lax.axis_index("c")
    sid = lax.axis_index("s")
    wid = sid * _NC + cid

    def zbody(i, carry):
      ch = i * _NS + sid

      @pl.when(ch < s_nch)
      def _():
        base = pl.multiple_of(ch * _C, 8)
        pltpu.sync_copy(zeros, shared.at[pl.ds(base, _C)])

      return carry

    lax.fori_loop(0, s_ceil, zbody, 0)
    if ones_mode:
      pltpu.sync_copy(vals, vals_v)
    plsc.subcore_barrier()

    def body(i, carry):
      ch = i * _NW + wid

      @pl.when(ch < nch)
      def _():
        base = pl.multiple_of(ch * _C, 8)
        pltpu.sync_copy(idx.at[pl.ds(base, _C)], idx_v)
        if not ones_mode:
          pltpu.sync_copy(vals.at[pl.ds(base, _C)], vals_v)
        pltpu.sync_copy(vals_v, shared.at[idx_v], add=True)

      return carry

    lax.fori_loop(0, ceil, body, 0)
    plsc.subcore_barrier()

    def obody(i, carry):
      ch = i * _NS + sid

      @pl.when(ch < s_nch)
      def _():
        base = pl.multiple_of(ch * _C, 8)
        pltpu.sync_copy(shared.at[pl.ds(base, _C)], out.at[cid, pl.ds(base, _C)])

      return carry

    lax.fori_loop(0, s_ceil, obody, 0)

  return k


_DOT = dict(precision=lax.Precision.DEFAULT, preferred_element_type=jnp.float32)


def _onehot(ids2d, B):
  return (ids2d == lax.broadcasted_iota(jnp.int32, (1, B), 1)).astype(jnp.float32)


def _edge_mlp_body(relu_ea, B,
                   gxs, gxd, ea, ids, fid, w1a, w1b, w1c, w1d, b1, w2, b2,
                   ue, esum, ecnt):
  pid = pl.program_id(0)

  @pl.when(pid == 0)
  def _():
    esum[...] = jnp.zeros_like(esum)
    ecnt[...] = jnp.zeros_like(ecnt)

  ea_t = ea[...]
  if relu_ea:
    ea_t = jnp.maximum(ea_t, 0.0)
  oh = _onehot(ids[...], B)
  t = jnp.dot(fid[...], w1d[...], **_DOT)
  h = (jnp.dot(gxs[...], w1a[...], **_DOT)
       + jnp.dot(gxd[...], w1b[...], **_DOT)
       + jnp.dot(ea_t, w1c[...], **_DOT)
       + jnp.dot(oh, t, **_DOT) + b1[...])
  h = jnp.maximum(h, 0.0)
  u = jnp.dot(h, w2[...], **_DOT) + b2[...]
  ue[...] = u
  cdims = (((0,), (0,)), ((), ()))
  esum[...] += lax.dot_general(oh, u, cdims, **_DOT)
  ecnt[...] += lax.dot_general(oh, jnp.ones_like(ids[...], jnp.float32), cdims,
                               **_DOT)


def _edge_mlp(gxs, gxd, ea, ids, fid, w1a, w1b, w1c, w1d, b1, w2, b2,
              relu_ea, tile):
  E = gxs.shape[0]
  Dea = ea.shape[1]
  B = fid.shape[0]
  grid = E // tile
  full = lambda a: pl.BlockSpec(a.shape, lambda i: (0,) * a.ndim)
  return pl.pallas_call(
      functools.partial(_edge_mlp_body, relu_ea, B),
      grid=(grid,),
      in_specs=[
          pl.BlockSpec((tile, 128), lambda i: (i, 0)),
          pl.BlockSpec((tile, 128), lambda i: (i, 0)),
          pl.BlockSpec((tile, Dea), lambda i: (i, 0)),
          pl.BlockSpec((tile, 1), lambda i: (i, 0)),
          full(fid), full(w1a), full(w1b), full(w1c), full(w1d), full(b1),
          full(w2), full(b2),
      ],
      out_specs=[
          pl.BlockSpec((tile, 128), lambda i: (i, 0)),
          pl.BlockSpec((B, 128), lambda i: (0, 0)),
          pl.BlockSpec((B, 1), lambda i: (0, 0)),
      ],
      out_shape=[
          jax.ShapeDtypeStruct((E, 128), jnp.float32),
          jax.ShapeDtypeStruct((B, 128), jnp.float32),
          jax.ShapeDtypeStruct((B, 1), jnp.float32),
      ],
  )(gxs, gxd, ea, ids, fid, w1a, w1b, w1c, w1d, b1, w2, b2)


def _node_mlp_body(B, sums, cnts, xn, ids, fid, w1a, w1b, w1c, b1, w2, b2,
                   xnext, nsum, ncnt):
  pid = pl.program_id(0)

  @pl.when(pid == 0)
  def _():
    nsum[...] = jnp.zeros_like(nsum)
    ncnt[...] = jnp.zeros_like(ncnt)

  avg = sums[...] / jnp.maximum(cnts[...], 1.0)
  oh = _onehot(ids[...], B)
  t = jnp.dot(fid[...], w1c[...], **_DOT)
  h = (jnp.dot(avg, w1a[...], **_DOT)
       + jnp.dot(xn[...], w1b[...], **_DOT)
       + jnp.dot(oh, t, **_DOT) + b1[...])
  h = jnp.maximum(h, 0.0)
  u = jnp.dot(h, w2[...], **_DOT) + b2[...]
  xnext[...] = jnp.maximum(u, 0.0)
  cdims = (((0,), (0,)), ((), ()))
  nsum[...] += lax.dot_general(oh, u, cdims, **_DOT)
  ncnt[...] += lax.dot_general(oh, jnp.ones_like(ids[...], jnp.float32), cdims,
                               **_DOT)


def _node_mlp(sums, cnts, xn, ids, fid, w1a, w1b, w1c, b1, w2, b2, tile):
  N = xn.shape[0]
  B = fid.shape[0]
  grid = N // tile
  full = lambda a: pl.BlockSpec(a.shape, lambda i: (0,) * a.ndim)
  return pl.pallas_call(
      functools.partial(_node_mlp_body, B),
      grid=(grid,),
      in_specs=[
          pl.BlockSpec((tile, 128), lambda i: (i, 0)),
          pl.BlockSpec((tile, 128), lambda i: (i, 0)),
          pl.BlockSpec((tile, 128), lambda i: (i, 0)),
          pl.BlockSpec((tile, 1), lambda i: (i, 0)),
          full(fid), full(w1a), full(w1b), full(w1c), full(b1), full(w2),
          full(b2),
      ],
      out_specs=[
          pl.BlockSpec((tile, 128), lambda i: (i, 0)),
          pl.BlockSpec((B, 128), lambda i: (0, 0)),
          pl.BlockSpec((B, 1), lambda i: (0, 0)),
      ],
      out_shape=[
          jax.ShapeDtypeStruct((N, 128), jnp.float32),
          jax.ShapeDtypeStruct((B, 128), jnp.float32),
          jax.ShapeDtypeStruct((B, 1), jnp.float32),
      ],
  )(sums, cnts, xn, ids, fid, w1a, w1b, w1c, b1, w2, b2)


def _fid_mlp_body(nsum, ncnt, esum, ecnt, fid, w1a, w1b, w1c, b1, w2, b2, out):
  avg_n = nsum[...] / jnp.maximum(ncnt[...], 1.0)
  avg_e = esum[...] / jnp.maximum(ecnt[...], 1.0)
  h = (jnp.dot(avg_n, w1a[...], **_DOT)
       + jnp.dot(avg_e, w1b[...], **_DOT)
       + jnp.dot(fid[...], w1c[...], **_DOT) + b1[...])
  h = jnp.maximum(h, 0.0)
  out[...] = jnp.dot(h, w2[...], **_DOT) + b2[...]


def _fid_mlp(nsum, ncnt, esum, ecnt, fid, w1a, w1b, w1c, b1, w2, b2):
  B = fid.shape[0]
  Do = w2.shape[1]
  return pl.pallas_call(
      _fid_mlp_body,
      out_shape=jax.ShapeDtypeStruct((B, Do), jnp.float32),
  )(nsum, ncnt, esum, ecnt, fid, w1a, w1b, w1c, b1, w2, b2)


def _pool_body(relu_in, B, vals, ids, psum, pcnt, pmax):
  pid = pl.program_id(0)

  @pl.when(pid == 0)
  def _():
    psum[...] = jnp.zeros_like(psum)
    pcnt[...] = jnp.zeros_like(pcnt)
    pmax[...] = jnp.full_like(pmax, -jnp.inf)

  v = vals[...]
  if relu_in:
    v = jnp.maximum(v, 0.0)
  ids_t = ids[...]
  oh = _onehot(ids_t, B)
  cdims = (((0,), (0,)), ((), ()))
  psum[...] += lax.dot_general(oh, v, cdims, **_DOT)
  pcnt[...] += lax.dot_general(oh, jnp.ones_like(ids_t, jnp.float32), cdims,
                               **_DOT)
  rows = []
  for b in range(B):
    mask = ids_t == b
    rows.append(jnp.max(jnp.where(mask, v, -jnp.inf), axis=0, keepdims=True))
  pmax[...] = jnp.maximum(pmax[...], jnp.concatenate(rows, axis=0))


def _pool(vals, ids, relu_in, B, tile):
  M = vals.shape[0]
  grid = M // tile
  return pl.pallas_call(
      functools.partial(_pool_body, relu_in, B),
      grid=(grid,),
      in_specs=[
          pl.BlockSpec((tile, 128), lambda i: (i, 0)),
          pl.BlockSpec((tile, 1), lambda i: (i, 0)),
      ],
      out_specs=[
          pl.BlockSpec((B, 128), lambda i: (0, 0)),
          pl.BlockSpec((B, 1), lambda i: (0, 0)),
          pl.BlockSpec((B, 128), lambda i: (0, 0)),
      ],
      out_shape=[
          jax.ShapeDtypeStruct((B, 128), jnp.float32),
          jax.ShapeDtypeStruct((B, 1), jnp.float32),
          jax.ShapeDtypeStruct((B, 128), jnp.float32),
      ],
  )(vals, ids)


def kernel(x, edge_attr, emb_table, params, edge_index, fidelity_indicator,
           batch_mapping):
  N = x.shape[0]
  E = edge_attr.shape[0]
  B = fidelity_indicator.shape[0]
  src = edge_index[0].astype(jnp.int32)
  dst = edge_index[1].astype(jnp.int32)
  ebatch = batch_mapping[src].astype(jnp.int32)
  batch2 = batch_mapping.astype(jnp.int32).reshape(N, 1)
  ebatch2 = ebatch.reshape(E, 1)
  fid = emb_table[fidelity_indicator]

  gather_n = _make_sc_gather(N, E, 128)
  scatter_n = _make_sc_scatter_add(E, N, 128)
  zeros_c = jnp.zeros((_C, 128), jnp.float32)

  scatter_ones = _make_sc_scatter_add(E, N, 128, ones_mode=True)
  cnt2 = scatter_ones(jnp.ones((_C, 128), jnp.float32), dst, zeros_c)
  cnts = cnt2[0] + cnt2[1]

  def split(pre, sizes):
    w1 = params[pre + "_W1"]
    parts = []
    o = 0
    for s in sizes:
      parts.append(w1[o:o + s])
      o += s
    return parts, params[pre + "_b1"].reshape(1, -1), params[pre + "_W2"], \
        params[pre + "_b2"].reshape(1, -1)

  xl = x
  el = edge_attr
  fl = fid
  f_raw = None
  ue = None
  for li, pre in enumerate(["m0", "m1", "m2"]):
    dea = el.shape[1]
    (w1a, w1b, w1c, w1d), b1e, w2e, b2e = split(pre + "_e",
                                                [128, 128, dea, 32])
    gxs = gather_n(xl, src)
    gxd = gather_n(xl, dst)
    ue, esum, ecnt = _edge_mlp(gxs, gxd, el, ebatch2, fl,
                               w1a, w1b, w1c, w1d, b1e, w2e, b2e,
                               relu_ea=(li > 0), tile=640)
    sum2 = scatter_n(ue, dst, zeros_c)
    sums = sum2[0] + sum2[1]
    (v1a, v1b, v1c), b1v, w2v, b2v = split(pre + "_v", [128, 128, 32])
    xl, nsum, ncnt = _node_mlp(sums, cnts, xl, batch2, fl,
                               v1a, v1b, v1c, b1v, w2v, b2v, tile=400)
    (u1a, u1b, u1c), b1u, w2u, b2u = split(pre + "_u", [128, 128, 32])
    f_raw = _fid_mlp(nsum, ncnt, esum, ecnt, fl,
                     u1a, u1b, u1c, b1u, w2u, b2u)
    fl = jnp.maximum(f_raw, 0.0)
    el = ue

  xs, xc, xm = _pool(xl, batch2, False, B, tile=400)
  es, ec, em = _pool(ue, ebatch2, True, B, tile=640)
  xmean = xs / jnp.maximum(xc, 1.0)
  emean = es / jnp.maximum(ec, 1.0)
  xm = jnp.where(xc > 0, xm, 0.0)
  em = jnp.where(ec > 0, em, 0.0)
  x_batched = jnp.concatenate([xs, xmean, xm], axis=1)
  e_batched = jnp.concatenate([es, emean, em], axis=1)
  return (x_batched, e_batched, fl)
